# baseline (speedup 1.0000x reference)
import functools

import jax
import jax.numpy as jnp
import numpy as np
from jax.experimental import pallas as pl
from jax.experimental.pallas import tpu as pltpu

_N, _L, _K = 128, 8192, 20
_BL = 2048
_SG = _BL // 128
_GL = _L // _BL
_TINY = np.float32(np.finfo(np.float32).tiny)
_LOG_EPS = np.float32(np.log(np.float64(np.float32(1e-8))))


def _threefry_bits(cnt):
    ks0 = np.uint32(0)
    ks1 = np.uint32(1)
    ks2 = np.uint32(0x1BD11BDB)
    ks = (ks0, ks1, ks2)
    rot = (13, 15, 26, 6, 17, 29, 16, 24)

    x0 = jnp.zeros_like(cnt)
    x1 = cnt + ks1

    def rotl(v, d):
        return jax.lax.shift_left(v, np.uint32(d)) | jax.lax.shift_right_logical(
            v, np.uint32(32 - d))

    for i in range(5):
        rs = rot[:4] if i % 2 == 0 else rot[4:]
        for r in rs:
            x0 = x0 + x1
            x1 = rotl(x1, r)
            x1 = x0 ^ x1
        x0 = x0 + ks[(i + 1) % 3]
        x1 = x1 + ks[(i + 2) % 3] + np.uint32(i + 1)
    return x0 ^ x1


def _fused_kernel(t_ref, ab_ref, x0_lane_ref, m_lane_ref, x0_sub_ref, m_sub_ref,
                  c_ref, xt_ref):
    i = pl.program_id(0)
    n = i // _GL

    ab = ab_ref[t_ref[n]]
    q = (1.0 - ab) / 20.0
    a = ab + q

    x0s = x0_sub_ref[:, :]
    ms = m_sub_ref[:, :] != 0
    kio2 = jax.lax.broadcasted_iota(jnp.int32, (_BL, _K), 1)
    oh2 = x0s == kio2
    hi = jnp.where(ms, a, 1.0).astype(jnp.float32)
    lo = jnp.where(ms, q, 0.0).astype(jnp.float32)
    c_ref[:, :] = jnp.where(oh2, hi, lo)

    x0l = x0_lane_ref[:, :].reshape(_SG, 1, 128)
    ml = m_lane_ref[:, :].reshape(_SG, 1, 128) != 0
    kio = jax.lax.broadcasted_iota(jnp.int32, (_SG, _K, 128), 1)
    ohl = x0l == kio
    c_like = jnp.where(
        ml, jnp.where(ohl, a, q), jnp.where(ohl, 1.0, 0.0)).astype(jnp.float32)
    logits = jnp.log(c_like + 1e-8)

    sio = jax.lax.broadcasted_iota(jnp.uint32, (_SG, _K, 128), 0)
    lio = jax.lax.broadcasted_iota(jnp.uint32, (_SG, _K, 128), 2)
    base = jnp.uint32(i * (_BL * _K))
    cnt = base + sio * np.uint32(128 * _K) + lio * np.uint32(_K) + kio.astype(jnp.uint32)
    bits = _threefry_bits(cnt)
    fb = jax.lax.shift_right_logical(bits, np.uint32(9)) | np.uint32(0x3F800000)
    f = jax.lax.bitcast_convert_type(fb, jnp.float32) - 1.0
    u = jnp.maximum(_TINY, f + _TINY)
    g = -jnp.log(-jnp.log(u))

    s = logits + g
    xt = jnp.argmax(s, axis=1)
    xt_ref[:, :] = xt.astype(jnp.int32)


@jax.jit
def kernel(x_0, mask_generate, t, alpha_bars):
    m_i32 = mask_generate.astype(jnp.int32)
    grid = (_N * _GL,)

    c_t, x_t = pl.pallas_call(
        _fused_kernel,
        grid=grid,
        in_specs=[
            pl.BlockSpec(memory_space=pltpu.SMEM),
            pl.BlockSpec(memory_space=pltpu.SMEM),
            pl.BlockSpec((_SG, 128), lambda i: (i, 0)),
            pl.BlockSpec((_SG, 128), lambda i: (i, 0)),
            pl.BlockSpec((_BL, 1), lambda i: (i, 0)),
            pl.BlockSpec((_BL, 1), lambda i: (i, 0)),
        ],
        out_specs=[
            pl.BlockSpec((_BL, _K), lambda i: (i, 0)),
            pl.BlockSpec((_SG, 128), lambda i: (i, 0)),
        ],
        out_shape=[
            jax.ShapeDtypeStruct((_N * _L, _K), jnp.float32),
            jax.ShapeDtypeStruct((_N * _L // 128, 128), jnp.int32),
        ],
        compiler_params=pltpu.CompilerParams(
            dimension_semantics=("arbitrary",),
        ),
    )(
        t.astype(jnp.int32),
        alpha_bars,
        x_0.reshape(_N * _L // 128, 128),
        m_i32.reshape(_N * _L // 128, 128),
        x_0.reshape(_N * _L, 1),
        m_i32.reshape(_N * _L, 1),
    )
    return c_t.reshape(_N, _L, _K), x_t.reshape(_N, _L)


# speedup vs baseline: 2.4281x; 2.4281x over previous
import jax
import jax.numpy as jnp
import numpy as np
from jax.experimental import pallas as pl
from jax.experimental.pallas import tpu as pltpu

_N, _L, _K = 128, 8192, 20
_BL = 2048
_SG = _BL // 128
_GL = _L // _BL
_TINY = np.float32(np.finfo(np.float32).tiny)


def _threefry_bits(cnt):
    ks = (np.uint32(0), np.uint32(1), np.uint32(0x1BD11BDB))
    rot = (13, 15, 26, 6, 17, 29, 16, 24)

    x0 = jnp.zeros_like(cnt)
    x1 = cnt + ks[1]

    def rotl(v, d):
        return jax.lax.shift_left(v, np.uint32(d)) | jax.lax.shift_right_logical(
            v, np.uint32(32 - d))

    for i in range(5):
        rs = rot[:4] if i % 2 == 0 else rot[4:]
        for r in rs:
            x0 = x0 + x1
            x1 = rotl(x1, r)
            x1 = x0 ^ x1
        x0 = x0 + ks[(i + 1) % 3]
        x1 = x1 + ks[(i + 2) % 3] + np.uint32(i + 1)
    return x0 ^ x1


def _fused_kernel(t_ref, ab_ref, x0_ref, m_ref, c_ref, xt_ref):
    i = pl.program_id(0)
    n = i // _GL

    ab = ab_ref[t_ref[n]]
    q = (1.0 - ab) / 20.0
    a = ab + q

    kio = jax.lax.broadcasted_iota(jnp.int32, (_K, 128), 0)
    lane20 = jax.lax.broadcasted_iota(jnp.uint32, (_K, 128), 1) * np.uint32(_K) \
        + kio.astype(jnp.uint32)
    base = jnp.uint32(i * (_BL * _K))

    for s in range(_SG):
        x0s = jnp.broadcast_to(x0_ref[s, :][None, :], (_K, 128))
        ms = jnp.broadcast_to(m_ref[s, :][None, :], (_K, 128)) != 0
        oh = x0s == kio
        c_like = jnp.where(
            ms, jnp.where(oh, a, q), jnp.where(oh, 1.0, 0.0)).astype(jnp.float32)

        c_ref[s * 128:(s + 1) * 128, :] = c_like.T

        logits = jnp.log(c_like + 1e-8)

        cnt = (base + np.uint32(s * 128 * _K)) + lane20
        bits = _threefry_bits(cnt)
        fb = jax.lax.shift_right_logical(bits, np.uint32(9)) | np.uint32(0x3F800000)
        f = jax.lax.bitcast_convert_type(fb, jnp.float32) - 1.0
        u = jnp.maximum(_TINY, f + _TINY)
        g = -jnp.log(-jnp.log(u))

        s_val = logits + g
        xt_ref[s, :] = jnp.argmax(s_val, axis=0).astype(jnp.int32)


@jax.jit
def kernel(x_0, mask_generate, t, alpha_bars):
    m_i32 = mask_generate.astype(jnp.int32)
    grid = (_N * _GL,)

    c_t, x_t = pl.pallas_call(
        _fused_kernel,
        grid=grid,
        in_specs=[
            pl.BlockSpec(memory_space=pltpu.SMEM),
            pl.BlockSpec(memory_space=pltpu.SMEM),
            pl.BlockSpec((_SG, 128), lambda i: (i, 0)),
            pl.BlockSpec((_SG, 128), lambda i: (i, 0)),
        ],
        out_specs=[
            pl.BlockSpec((_BL, _K), lambda i: (i, 0)),
            pl.BlockSpec((_SG, 128), lambda i: (i, 0)),
        ],
        out_shape=[
            jax.ShapeDtypeStruct((_N * _L, _K), jnp.float32),
            jax.ShapeDtypeStruct((_N * _L // 128, 128), jnp.int32),
        ],
        compiler_params=pltpu.CompilerParams(
            dimension_semantics=("parallel",),
        ),
    )(
        t.astype(jnp.int32),
        alpha_bars,
        x_0.reshape(_N * _L // 128, 128),
        m_i32.reshape(_N * _L // 128, 128),
    )
    return c_t.reshape(_N, _L, _K), x_t.reshape(_N, _L)
